# baseline (speedup 1.0000x reference)
import jax
import jax.numpy as jnp
from jax import lax
from jax.experimental import pallas as pl
from jax.experimental.pallas import tpu as pltpu
from jax.experimental.pallas import tpu_sc as plsc

_NW = 32
_LANES = 16
_ROWS = 16
_COLS = 128
_PER_W = _ROWS * _COLS


def _sc_loss_body(flat_ref, idx_ref, tgt_ref, msk_ref, out_ref,
                  idx_v, val_v, tgt_v, msk_v, acc_v, sem):
    wid = lax.axis_index("s") * 2 + lax.axis_index("c")
    pltpu.sync_copy(idx_ref.at[wid], idx_v)
    pltpu.sync_copy(tgt_ref.at[wid], tgt_v)
    pltpu.sync_copy(msk_ref.at[wid], msk_v)

    copies = [
        pltpu.async_copy(flat_ref.at[idx_v.at[j]], val_v.at[j], sem)
        for j in range(_ROWS)
    ]
    for cp in copies:
        cp.wait()

    def body(i, carry):
        acc_l, acc_m = carry
        r = i // (_COLS // _LANES)
        col = (i % (_COLS // _LANES)) * _LANES
        v = val_v[r, pl.ds(col, _LANES)]
        t = tgt_v[r, pl.ds(col, _LANES)]
        m = msk_v[r, pl.ds(col, _LANES)]
        a = jnp.abs(v - t) * m
        loss = jnp.where(a < 1.0, 0.5 * a * a, a - 0.5)
        return acc_l + loss, acc_m + m

    zero = jnp.zeros((_LANES,), jnp.float32)
    acc_l, acc_m = lax.fori_loop(0, _PER_W // _LANES, body, (zero, zero))
    acc_v[0, :] = acc_l
    acc_v[1, :] = acc_m
    pltpu.sync_copy(acc_v, out_ref.at[wid])


def kernel(output, mask, ind, target):
    B, dim, H, W = output.shape
    M = ind.shape[1]
    HW = H * W
    N = B * M * dim
    NPAD = _NW * _PER_W

    flat = output.reshape(-1)
    base = (jnp.arange(B, dtype=jnp.int32) * (dim * HW))[:, None, None]
    doff = (jnp.arange(dim, dtype=jnp.int32) * HW)[None, None, :]
    gidx = (base + doff + ind[:, :, None]).reshape(-1)
    mf = jnp.broadcast_to(mask[:, :, None], (B, M, dim))
    mf = mf.astype(jnp.float32).reshape(-1)
    tf = target.reshape(-1)

    pad = NPAD - N
    gidx = jnp.concatenate([gidx, jnp.zeros((pad,), jnp.int32)])
    mf = jnp.concatenate([mf, jnp.zeros((pad,), jnp.float32)])
    tf = jnp.concatenate([tf, jnp.zeros((pad,), jnp.float32)])
    gidx = gidx.reshape(_NW, _ROWS, _COLS)
    mf = mf.reshape(_NW, _ROWS, _COLS)
    tf = tf.reshape(_NW, _ROWS, _COLS)

    mesh = plsc.VectorSubcoreMesh(core_axis_name="c", subcore_axis_name="s")
    fn = pl.kernel(
        _sc_loss_body,
        out_type=jax.ShapeDtypeStruct((_NW, 2, _LANES), jnp.float32),
        mesh=mesh,
        scratch_types=[
            pltpu.VMEM((_ROWS, _COLS), jnp.int32),
            pltpu.VMEM((_ROWS, _COLS), jnp.float32),
            pltpu.VMEM((_ROWS, _COLS), jnp.float32),
            pltpu.VMEM((_ROWS, _COLS), jnp.float32),
            pltpu.VMEM((2, _LANES), jnp.float32),
            pltpu.SemaphoreType.DMA,
        ],
    )
    parts = fn(flat, gidx, tf, mf)
    loss = parts[:, 0, :].sum()
    num = parts[:, 1, :].sum() / dim
    return loss / (num + 1e-4)


# speedup vs baseline: 1.6589x; 1.6589x over previous
import dataclasses

import jax
import jax.numpy as jnp
from jax import lax
from jax.experimental import pallas as pl
from jax.experimental.pallas import tpu as pltpu
from jax.experimental.pallas import tpu_sc as plsc

_NW = 32
_LANES = 16
_ROWS = 16
_COLS = 128


def _make_body(B, dim, HW, M):
    PP = B * M // _NW
    EPW = PP * dim
    NCH = EPW // _LANES
    SB = dim * HW
    assert dim == 2 and PP == 2 * M and NCH <= _ROWS * (_COLS // _LANES)

    def body(flat_ref, ind_ref, tgt_ref, msk_ref, out_ref,
             ind_v, msk_v, idx_v, val_v, tgt_v, acc_v,
             sem_t, sem_m, sem_g):
        wid = lax.axis_index("s") * 2 + lax.axis_index("c")
        tgt_cp = pltpu.async_copy(
            tgt_ref.at[pl.ds(wid * EPW, EPW)], tgt_v.at[pl.ds(0, EPW)], sem_t)
        msk_cp = pltpu.async_copy(
            msk_ref.at[pl.ds(wid * PP, PP)], msk_v, sem_m)
        pltpu.sync_copy(ind_ref.at[pl.ds(wid * PP, PP)], ind_v)

        iota = lax.broadcasted_iota(jnp.int32, (_LANES,), 0)
        half = lax.shift_right_logical(iota, 1)
        d_off = (iota & 1) * HW
        base = wid * (2 * SB)

        zero = jnp.zeros((_LANES,), jnp.int32)
        for c in range(NCH, _ROWS * (_COLS // _LANES)):
            idx_v[c * _LANES // _COLS, pl.ds(c * _LANES % _COLS, _LANES)] = zero

        gcopies = []
        for j in range(_ROWS):
            for k in range(_COLS // _LANES):
                c = j * (_COLS // _LANES) + k
                if c >= NCH:
                    break
                p_loc = half + (c * (_COLS // _LANES))
                ind_g = plsc.load_gather(ind_v, [p_loc])
                b_off = jnp.where(p_loc >= M, SB, 0)
                idx_v[j, pl.ds(k * _LANES, _LANES)] = base + b_off + d_off + ind_g
            gcopies.append(pltpu.async_copy(
                flat_ref.at[idx_v.at[j]], val_v.at[j], sem_g))
        for cp in gcopies:
            cp.wait()
        tgt_cp.wait()
        msk_cp.wait()

        acc_l = jnp.zeros((_LANES,), jnp.float32)
        acc_m = jnp.zeros((_LANES,), jnp.float32)
        for c in range(NCH):
            v = val_v[c * _LANES // _COLS, pl.ds(c * _LANES % _COLS, _LANES)]
            t = tgt_v[pl.ds(c * _LANES, _LANES)]
            p_loc = half + (c * (_COLS // _LANES))
            m = plsc.load_gather(msk_v, [p_loc]).astype(jnp.float32)
            a = jnp.abs(v - t) * m
            acc_l = acc_l + jnp.where(a < 1.0, 0.5 * a * a, a - 0.5)
            acc_m = acc_m + m
        acc_v[0, :] = acc_l
        acc_v[1, :] = acc_m
        pltpu.sync_copy(acc_v, out_ref.at[wid])

    return body


def kernel(output, mask, ind, target):
    B, dim, H, W = output.shape
    M = ind.shape[1]
    HW = H * W
    PP = B * M // _NW
    EPW = PP * dim

    cp = pltpu.CompilerParams()
    if "needs_layout_passes" in pltpu.CompilerParams.__dataclass_fields__:
        cp = dataclasses.replace(cp, needs_layout_passes=False)
    mesh = plsc.VectorSubcoreMesh(core_axis_name="c", subcore_axis_name="s")
    fn = pl.kernel(
        _make_body(B, dim, HW, M),
        out_type=jax.ShapeDtypeStruct((_NW, 2, _LANES), jnp.float32),
        mesh=mesh,
        compiler_params=cp,
        scratch_types=[
            pltpu.VMEM((PP,), jnp.int32),
            pltpu.VMEM((PP,), jnp.int32),
            pltpu.VMEM((_ROWS, _COLS), jnp.int32),
            pltpu.VMEM((_ROWS, _COLS), jnp.float32),
            pltpu.VMEM((_ROWS * _COLS,), jnp.float32),
            pltpu.VMEM((2, _LANES), jnp.float32),
            pltpu.SemaphoreType.DMA,
            pltpu.SemaphoreType.DMA,
            pltpu.SemaphoreType.DMA,
        ],
    )
    parts = fn(output.reshape(-1), ind.reshape(-1), target.reshape(-1),
               mask.reshape(-1))
    loss = parts[:, 0, :].sum()
    num = parts[:, 1, :].sum() / dim
    return loss / (num + 1e-4)


# speedup vs baseline: 1.9098x; 1.1512x over previous
import dataclasses

import jax
import jax.numpy as jnp
from jax import lax
from jax.experimental import pallas as pl
from jax.experimental.pallas import tpu as pltpu
from jax.experimental.pallas import tpu_sc as plsc

_NW = 32
_LANES = 16
_ROWS = 16
_COLS = 128


def _make_body(B, dim, HW, M):
    PP = B * M // _NW
    EPW = PP * dim
    NCH = EPW // _LANES
    SB = dim * HW
    assert dim == 2 and PP == 2 * M and NCH <= _ROWS * (_COLS // _LANES)

    def body(flat_ref, ind_ref, tgt_ref, msk_ref, out_ref,
             ind_v, msk_v, feat_v, tgt_v, acc_v,
             sem_f, sem_t, sem_m):
        wid = lax.axis_index("s") * 2 + lax.axis_index("c")
        feat_cp = pltpu.async_copy(
            flat_ref.at[pl.ds(wid * (2 * SB), 2 * SB)], feat_v, sem_f)
        tgt_cp = pltpu.async_copy(
            tgt_ref.at[pl.ds(wid * EPW, EPW)], tgt_v, sem_t)
        msk_cp = pltpu.async_copy(
            msk_ref.at[pl.ds(wid * PP, PP)], msk_v, sem_m)
        pltpu.sync_copy(ind_ref.at[pl.ds(wid * PP, PP)], ind_v)

        iota = lax.broadcasted_iota(jnp.int32, (_LANES,), 0)
        half = lax.shift_right_logical(iota, 1)
        d_off = (iota & 1) * HW
        tgt_cp.wait()
        msk_cp.wait()
        feat_cp.wait()

        acc_l = jnp.zeros((_LANES,), jnp.float32)
        acc_m = jnp.zeros((_LANES,), jnp.float32)
        for c in range(NCH):
            p_loc = half + (c * (_LANES // dim))
            ind_g = plsc.load_gather(ind_v, [p_loc])
            b_off = jnp.where(p_loc >= M, SB, 0)
            v = plsc.load_gather(feat_v, [b_off + d_off + ind_g])
            t = tgt_v[pl.ds(c * _LANES, _LANES)]
            m = plsc.load_gather(msk_v, [p_loc]).astype(jnp.float32)
            a = jnp.abs(v - t) * m
            acc_l = acc_l + jnp.where(a < 1.0, 0.5 * a * a, a - 0.5)
            acc_m = acc_m + m
        acc_v[0, :] = acc_l
        acc_v[1, :] = acc_m
        pltpu.sync_copy(acc_v, out_ref.at[wid])

    return body


def kernel(output, mask, ind, target):
    B, dim, H, W = output.shape
    M = ind.shape[1]
    HW = H * W
    PP = B * M // _NW
    EPW = PP * dim

    cp = pltpu.CompilerParams()
    if "needs_layout_passes" in pltpu.CompilerParams.__dataclass_fields__:
        cp = dataclasses.replace(cp, needs_layout_passes=False)
    mesh = plsc.VectorSubcoreMesh(core_axis_name="c", subcore_axis_name="s")
    fn = pl.kernel(
        _make_body(B, dim, HW, M),
        out_type=jax.ShapeDtypeStruct((_NW, 2, _LANES), jnp.float32),
        mesh=mesh,
        compiler_params=cp,
        scratch_types=[
            pltpu.VMEM((PP,), jnp.int32),
            pltpu.VMEM((PP,), jnp.int32),
            pltpu.VMEM((2 * dim * HW,), jnp.float32),
            pltpu.VMEM((EPW,), jnp.float32),
            pltpu.VMEM((2, _LANES), jnp.float32),
            pltpu.SemaphoreType.DMA,
            pltpu.SemaphoreType.DMA,
            pltpu.SemaphoreType.DMA,
        ],
    )
    parts = fn(output.reshape(-1), ind.reshape(-1), target.reshape(-1),
               mask.reshape(-1))
    loss = parts[:, 0, :].sum()
    num = parts[:, 1, :].sum() / dim
    return loss / (num + 1e-4)
